# baseline (speedup 1.0000x reference)
import functools
import math

import jax
import jax.numpy as jnp
from jax import lax
from jax.experimental import pallas as pl
from jax.experimental.pallas import tpu as pltpu
from jax.experimental.pallas import tpu_sc as plsc

_SHAPE = (2, 32, 16384, 20)
_B, _N, _P, _K = _SHAPE
_M = _B * _N * _P
_NC, _NS = 2, 16
_NW = _NC * _NS
_L = 16
_KEEP = _K // 2

_GROUPS = _N // 8
_COLS = _P // 2048
_UNITS = 2 * _B * _KEEP * _GROUPS * _COLS
_UPW = _UNITS // _NW


_NBUF = 4


def _make_plane_kernel():
    mesh = plsc.VectorSubcoreMesh(core_axis_name="c", subcore_axis_name="s")
    out_plane = jax.ShapeDtypeStruct((_B, _KEEP, _N, _P), jnp.uint32)

    @functools.partial(
        pl.kernel,
        mesh=mesh,
        out_type=(out_plane, out_plane),
        scratch_types=[
            [pltpu.VMEM((8, 2048), jnp.uint32) for _ in range(_NBUF)],
            [pltpu.SemaphoreType.DMA for _ in range(_NBUF)],
            [pltpu.SemaphoreType.DMA for _ in range(_NBUF)],
        ],
        compiler_params=pltpu.CompilerParams(use_tc_tiling_on_sc=True),
    )
    def plane_kernel(lo_hbm, hi_hbm, olo_hbm, ohi_hbm, bufs, sin, sout):
        wid = lax.axis_index("s") * _NC + lax.axis_index("c")
        upw = _UNITS // 2 // _NW
        base = wid * upw
        steps = []
        for src, dst in ((lo_hbm, olo_hbm), (hi_hbm, ohi_hbm)):
            for j in range(upw):
                q = base + j
                c = q % _COLS
                q = q // _COLS
                g = q % _GROUPS
                q = q // _GROUPS
                k = q % _KEEP
                i0 = q // _KEEP
                idx = (i0, k, pl.ds(8 * g, 8), pl.ds(2048 * c, 2048))
                steps.append((src, dst, idx))
        n = len(steps)
        lag = 2
        in_h = [None] * _NBUF
        out_h = [None] * _NBUF
        for u in range(n + lag):
            if u < n:
                b = u % _NBUF
                if u >= _NBUF:
                    out_h[b].wait()
                src, _, idx = steps[u]
                in_h[b] = pltpu.async_copy(src.at[idx], bufs[b], sin[b])
            v = u - lag
            if v >= 0:
                bv = v % _NBUF
                in_h[bv].wait()
                _, dst, idx = steps[v]
                out_h[bv] = pltpu.async_copy(bufs[bv], dst.at[idx], sout[bv])
        for v in range(max(0, n - _NBUF + lag), n):
            out_h[v % _NBUF].wait()

    return plane_kernel


_R = 1024
_RPW = _M // _NW
_CHUNKS = _RPW // _R


def _compaction_plan(w):
    rec = 2 * w
    p_in = rec * 32 // math.gcd(rec, 32)
    p_out = p_in // 2
    specs = []
    for j in range(p_out // _L):
        runs = []
        cur = None
        for d in range(_L):
            u = j * _L + d
            s = (u // w) * rec + (u % w)
            sv, sl = s // _L, s % _L
            shift = (sl - d) % _L
            if cur is not None and cur[2] == sv and cur[3] == shift:
                cur = (cur[0], d + 1, sv, shift)
            else:
                if cur is not None:
                    runs.append(cur)
                cur = (d, d + 1, sv, shift)
        runs.append(cur)
        specs.append(runs)
    return p_in, p_out, specs


def _rot(v, idx):
    return lax.gather(
        v,
        idx[:, None],
        dimension_numbers=lax.GatherDimensionNumbers(
            offset_dims=(),
            collapsed_slice_dims=(0,),
            start_index_map=(0,),
        ),
        slice_sizes=(1,),
        mode=lax.GatherScatterMode.PROMISE_IN_BOUNDS,
    )


def _make_compact_kernel(w):
    p_in, p_out, specs = _compaction_plan(w)
    rec = 2 * w
    in_words = _R * rec
    out_words = _R * w
    periods = in_words // p_in
    needed = sorted({r[2] for runs in specs for r in runs})
    shifts = sorted({r[3] for runs in specs for r in runs if r[3]})
    mesh = plsc.VectorSubcoreMesh(core_axis_name="c", subcore_axis_name="s")

    @functools.partial(
        pl.kernel,
        mesh=mesh,
        out_type=jax.ShapeDtypeStruct((_M * w,), jnp.int32),
        scratch_types=[
            pltpu.VMEM((in_words,), jnp.int32),
            pltpu.VMEM((out_words,), jnp.int32),
        ],
    )
    def compact_kernel(x_hbm, o_hbm, ibuf, obuf):
        wid = lax.axis_index("s") * _NC + lax.axis_index("c")
        iota = lax.broadcasted_iota(jnp.int32, (_L,), 0)
        rot_idx = {sh: (iota + sh) & (_L - 1) for sh in shifts}
        ge_mask = {
            lo: iota >= lo for runs in specs for (lo, _, _, _) in runs[1:]
        }
        in_base = wid * _RPW * rec
        out_base = wid * _RPW * w
        for t in range(_CHUNKS):
            pltpu.sync_copy(
                x_hbm.at[pl.ds(in_base + t * in_words, in_words)], ibuf
            )

            @plsc.parallel_loop(
                jnp.int32(0), jnp.int32(periods), jnp.int32(1), unroll=4
            )
            def _(p):
                b = p * p_in
                ob = p * p_out
                loads = {sv: ibuf[pl.ds(b + sv * _L, _L)] for sv in needed}
                rots = {}
                for runs in specs:
                    for _, _, sv, sh in runs:
                        if (sv, sh) not in rots:
                            v = loads[sv]
                            if sh:
                                v = _rot(v, rot_idx[sh])
                            rots[(sv, sh)] = v
                for j, runs in enumerate(specs):
                    val = rots[(runs[0][2], runs[0][3])]
                    for lo, _, sv, sh in runs[1:]:
                        val = jnp.where(ge_mask[lo], rots[(sv, sh)], val)
                    obuf[pl.ds(ob + j * _L, _L)] = val

            pltpu.sync_copy(
                obuf, o_hbm.at[pl.ds(out_base + t * out_words, out_words)]
            )

    return compact_kernel


def kernel(edge_index):
    if edge_index.dtype == jnp.int64:
        lo = edge_index.astype(jnp.uint32)
        hi = (edge_index >> 32).astype(jnp.uint32)
        lo_t = jnp.transpose(lo, (0, 3, 1, 2))
        hi_t = jnp.transpose(hi, (0, 3, 1, 2))
        olo, ohi = _make_plane_kernel()(lo_t, hi_t)
        olo = jnp.transpose(olo, (0, 2, 3, 1))
        ohi = jnp.transpose(ohi, (0, 2, 3, 1))
        out = (ohi.astype(jnp.uint64) << jnp.uint64(32)) | olo.astype(
            jnp.uint64
        )
        return out.astype(jnp.int64)
    w = _KEEP
    out = _make_compact_kernel(w)(edge_index.reshape(_M * 2 * w))
    out = out.reshape(_B, _N, _P, _KEEP)
    return out.astype(edge_index.dtype)


# speedup vs baseline: 1.0001x; 1.0001x over previous
import functools
import math

import jax
import jax.numpy as jnp
from jax import lax
from jax.experimental import pallas as pl
from jax.experimental.pallas import tpu as pltpu
from jax.experimental.pallas import tpu_sc as plsc

_SHAPE = (2, 32, 16384, 20)
_B, _N, _P, _K = _SHAPE
_M = _B * _N * _P
_NC, _NS = 2, 16
_NW = _NC * _NS
_L = 16
_KEEP = _K // 2

_GROUPS = _N // 8
_COLS = _P // 4096
_UNITS = 2 * _B * _KEEP * _GROUPS * _COLS
_UPW = _UNITS // _NW


_NBUF = 3


def _make_plane_kernel():
    mesh = plsc.VectorSubcoreMesh(core_axis_name="c", subcore_axis_name="s")
    out_plane = jax.ShapeDtypeStruct((_B, _KEEP, _N, _P), jnp.uint32)

    @functools.partial(
        pl.kernel,
        mesh=mesh,
        out_type=(out_plane, out_plane),
        scratch_types=[
            [pltpu.VMEM((8, 4096), jnp.uint32) for _ in range(_NBUF)],
            [pltpu.SemaphoreType.DMA for _ in range(_NBUF)],
            [pltpu.SemaphoreType.DMA for _ in range(_NBUF)],
        ],
        compiler_params=pltpu.CompilerParams(use_tc_tiling_on_sc=True),
    )
    def plane_kernel(lo_hbm, hi_hbm, olo_hbm, ohi_hbm, bufs, sin, sout):
        wid = lax.axis_index("s") * _NC + lax.axis_index("c")
        upw = _UNITS // 2 // _NW
        base = wid * upw
        steps = []
        for src, dst in ((lo_hbm, olo_hbm), (hi_hbm, ohi_hbm)):
            for j in range(upw):
                q = base + j
                c = q % _COLS
                q = q // _COLS
                g = q % _GROUPS
                q = q // _GROUPS
                k = q % _KEEP
                i0 = q // _KEEP
                idx = (i0, k, pl.ds(8 * g, 8), pl.ds(4096 * c, 4096))
                steps.append((src, dst, idx))
        n = len(steps)
        lag = 2
        in_h = [None] * _NBUF
        out_h = [None] * _NBUF
        for u in range(n + lag):
            if u < n:
                b = u % _NBUF
                if u >= _NBUF:
                    out_h[b].wait()
                src, _, idx = steps[u]
                in_h[b] = pltpu.async_copy(src.at[idx], bufs[b], sin[b])
            v = u - lag
            if v >= 0:
                bv = v % _NBUF
                in_h[bv].wait()
                _, dst, idx = steps[v]
                out_h[bv] = pltpu.async_copy(bufs[bv], dst.at[idx], sout[bv])
        for v in range(max(0, n - _NBUF + lag), n):
            out_h[v % _NBUF].wait()

    return plane_kernel


_R = 1024
_RPW = _M // _NW
_CHUNKS = _RPW // _R


def _compaction_plan(w):
    rec = 2 * w
    p_in = rec * 32 // math.gcd(rec, 32)
    p_out = p_in // 2
    specs = []
    for j in range(p_out // _L):
        runs = []
        cur = None
        for d in range(_L):
            u = j * _L + d
            s = (u // w) * rec + (u % w)
            sv, sl = s // _L, s % _L
            shift = (sl - d) % _L
            if cur is not None and cur[2] == sv and cur[3] == shift:
                cur = (cur[0], d + 1, sv, shift)
            else:
                if cur is not None:
                    runs.append(cur)
                cur = (d, d + 1, sv, shift)
        runs.append(cur)
        specs.append(runs)
    return p_in, p_out, specs


def _rot(v, idx):
    return lax.gather(
        v,
        idx[:, None],
        dimension_numbers=lax.GatherDimensionNumbers(
            offset_dims=(),
            collapsed_slice_dims=(0,),
            start_index_map=(0,),
        ),
        slice_sizes=(1,),
        mode=lax.GatherScatterMode.PROMISE_IN_BOUNDS,
    )


def _make_compact_kernel(w):
    p_in, p_out, specs = _compaction_plan(w)
    rec = 2 * w
    in_words = _R * rec
    out_words = _R * w
    periods = in_words // p_in
    needed = sorted({r[2] for runs in specs for r in runs})
    shifts = sorted({r[3] for runs in specs for r in runs if r[3]})
    mesh = plsc.VectorSubcoreMesh(core_axis_name="c", subcore_axis_name="s")

    @functools.partial(
        pl.kernel,
        mesh=mesh,
        out_type=jax.ShapeDtypeStruct((_M * w,), jnp.int32),
        scratch_types=[
            pltpu.VMEM((in_words,), jnp.int32),
            pltpu.VMEM((out_words,), jnp.int32),
        ],
    )
    def compact_kernel(x_hbm, o_hbm, ibuf, obuf):
        wid = lax.axis_index("s") * _NC + lax.axis_index("c")
        iota = lax.broadcasted_iota(jnp.int32, (_L,), 0)
        rot_idx = {sh: (iota + sh) & (_L - 1) for sh in shifts}
        ge_mask = {
            lo: iota >= lo for runs in specs for (lo, _, _, _) in runs[1:]
        }
        in_base = wid * _RPW * rec
        out_base = wid * _RPW * w
        for t in range(_CHUNKS):
            pltpu.sync_copy(
                x_hbm.at[pl.ds(in_base + t * in_words, in_words)], ibuf
            )

            @plsc.parallel_loop(
                jnp.int32(0), jnp.int32(periods), jnp.int32(1), unroll=4
            )
            def _(p):
                b = p * p_in
                ob = p * p_out
                loads = {sv: ibuf[pl.ds(b + sv * _L, _L)] for sv in needed}
                rots = {}
                for runs in specs:
                    for _, _, sv, sh in runs:
                        if (sv, sh) not in rots:
                            v = loads[sv]
                            if sh:
                                v = _rot(v, rot_idx[sh])
                            rots[(sv, sh)] = v
                for j, runs in enumerate(specs):
                    val = rots[(runs[0][2], runs[0][3])]
                    for lo, _, sv, sh in runs[1:]:
                        val = jnp.where(ge_mask[lo], rots[(sv, sh)], val)
                    obuf[pl.ds(ob + j * _L, _L)] = val

            pltpu.sync_copy(
                obuf, o_hbm.at[pl.ds(out_base + t * out_words, out_words)]
            )

    return compact_kernel


def kernel(edge_index):
    if edge_index.dtype == jnp.int64:
        lo = edge_index.astype(jnp.uint32)
        hi = (edge_index >> 32).astype(jnp.uint32)
        lo_t = jnp.transpose(lo, (0, 3, 1, 2))
        hi_t = jnp.transpose(hi, (0, 3, 1, 2))
        olo, ohi = _make_plane_kernel()(lo_t, hi_t)
        olo = jnp.transpose(olo, (0, 2, 3, 1))
        ohi = jnp.transpose(ohi, (0, 2, 3, 1))
        out = (ohi.astype(jnp.uint64) << jnp.uint64(32)) | olo.astype(
            jnp.uint64
        )
        return out.astype(jnp.int64)
    w = _KEEP
    out = _make_compact_kernel(w)(edge_index.reshape(_M * 2 * w))
    out = out.reshape(_B, _N, _P, _KEEP)
    return out.astype(edge_index.dtype)


# speedup vs baseline: 1.0078x; 1.0077x over previous
import functools
import math

import jax
import jax.numpy as jnp
from jax import lax
from jax.experimental import pallas as pl
from jax.experimental.pallas import tpu as pltpu
from jax.experimental.pallas import tpu_sc as plsc

_SHAPE = (2, 32, 16384, 20)
_B, _N, _P, _K = _SHAPE
_M = _B * _N * _P
_NC, _NS = 2, 16
_NW = _NC * _NS
_L = 16
_KEEP = _K // 2

_GROUPS = _N // 8
_COLS = _P // 4096
_UNITS = 2 * _B * _KEEP * _GROUPS * _COLS
_UPW = _UNITS // _NW


_NBUF = 3


def _make_plane_kernel():
    mesh = plsc.VectorSubcoreMesh(core_axis_name="c", subcore_axis_name="s")
    out_plane = jax.ShapeDtypeStruct((_B, _KEEP, _N, _P), jnp.uint32)

    @functools.partial(
        pl.kernel,
        mesh=mesh,
        out_type=out_plane,
        scratch_types=[
            [pltpu.VMEM((8, 4096), jnp.uint32) for _ in range(_NBUF)],
            [pltpu.SemaphoreType.DMA for _ in range(_NBUF)],
            [pltpu.SemaphoreType.DMA for _ in range(_NBUF)],
        ],
        compiler_params=pltpu.CompilerParams(use_tc_tiling_on_sc=True),
    )
    def plane_kernel(x_hbm, o_hbm, bufs, sin, sout):
        wid = lax.axis_index("s") * _NC + lax.axis_index("c")
        upw = _UNITS // 2 // _NW
        base = wid * upw
        steps = []
        for j in range(upw):
            q = base + j
            c = q % _COLS
            q = q // _COLS
            g = q % _GROUPS
            q = q // _GROUPS
            k = q % _KEEP
            i0 = q // _KEEP
            idx = (i0, k, pl.ds(8 * g, 8), pl.ds(4096 * c, 4096))
            steps.append((x_hbm, o_hbm, idx))
        n = len(steps)
        lag = 2
        in_h = [None] * _NBUF
        out_h = [None] * _NBUF
        for u in range(n + lag):
            if u < n:
                b = u % _NBUF
                if u >= _NBUF:
                    out_h[b].wait()
                src, _, idx = steps[u]
                in_h[b] = pltpu.async_copy(src.at[idx], bufs[b], sin[b])
            v = u - lag
            if v >= 0:
                bv = v % _NBUF
                in_h[bv].wait()
                _, dst, idx = steps[v]
                out_h[bv] = pltpu.async_copy(bufs[bv], dst.at[idx], sout[bv])
        for v in range(max(0, n - _NBUF + lag), n):
            out_h[v % _NBUF].wait()

    return plane_kernel


_R = 1024
_RPW = _M // _NW
_CHUNKS = _RPW // _R


def _compaction_plan(w):
    rec = 2 * w
    p_in = rec * 32 // math.gcd(rec, 32)
    p_out = p_in // 2
    specs = []
    for j in range(p_out // _L):
        runs = []
        cur = None
        for d in range(_L):
            u = j * _L + d
            s = (u // w) * rec + (u % w)
            sv, sl = s // _L, s % _L
            shift = (sl - d) % _L
            if cur is not None and cur[2] == sv and cur[3] == shift:
                cur = (cur[0], d + 1, sv, shift)
            else:
                if cur is not None:
                    runs.append(cur)
                cur = (d, d + 1, sv, shift)
        runs.append(cur)
        specs.append(runs)
    return p_in, p_out, specs


def _rot(v, idx):
    return lax.gather(
        v,
        idx[:, None],
        dimension_numbers=lax.GatherDimensionNumbers(
            offset_dims=(),
            collapsed_slice_dims=(0,),
            start_index_map=(0,),
        ),
        slice_sizes=(1,),
        mode=lax.GatherScatterMode.PROMISE_IN_BOUNDS,
    )


def _make_compact_kernel(w):
    p_in, p_out, specs = _compaction_plan(w)
    rec = 2 * w
    in_words = _R * rec
    out_words = _R * w
    periods = in_words // p_in
    needed = sorted({r[2] for runs in specs for r in runs})
    shifts = sorted({r[3] for runs in specs for r in runs if r[3]})
    mesh = plsc.VectorSubcoreMesh(core_axis_name="c", subcore_axis_name="s")

    @functools.partial(
        pl.kernel,
        mesh=mesh,
        out_type=jax.ShapeDtypeStruct((_M * w,), jnp.int32),
        scratch_types=[
            pltpu.VMEM((in_words,), jnp.int32),
            pltpu.VMEM((out_words,), jnp.int32),
        ],
    )
    def compact_kernel(x_hbm, o_hbm, ibuf, obuf):
        wid = lax.axis_index("s") * _NC + lax.axis_index("c")
        iota = lax.broadcasted_iota(jnp.int32, (_L,), 0)
        rot_idx = {sh: (iota + sh) & (_L - 1) for sh in shifts}
        ge_mask = {
            lo: iota >= lo for runs in specs for (lo, _, _, _) in runs[1:]
        }
        in_base = wid * _RPW * rec
        out_base = wid * _RPW * w
        for t in range(_CHUNKS):
            pltpu.sync_copy(
                x_hbm.at[pl.ds(in_base + t * in_words, in_words)], ibuf
            )

            @plsc.parallel_loop(
                jnp.int32(0), jnp.int32(periods), jnp.int32(1), unroll=4
            )
            def _(p):
                b = p * p_in
                ob = p * p_out
                loads = {sv: ibuf[pl.ds(b + sv * _L, _L)] for sv in needed}
                rots = {}
                for runs in specs:
                    for _, _, sv, sh in runs:
                        if (sv, sh) not in rots:
                            v = loads[sv]
                            if sh:
                                v = _rot(v, rot_idx[sh])
                            rots[(sv, sh)] = v
                for j, runs in enumerate(specs):
                    val = rots[(runs[0][2], runs[0][3])]
                    for lo, _, sv, sh in runs[1:]:
                        val = jnp.where(ge_mask[lo], rots[(sv, sh)], val)
                    obuf[pl.ds(ob + j * _L, _L)] = val

            pltpu.sync_copy(
                obuf, o_hbm.at[pl.ds(out_base + t * out_words, out_words)]
            )

    return compact_kernel


def kernel(edge_index):
    if edge_index.dtype == jnp.int64:
        lo = edge_index.astype(jnp.uint32)
        hi = (edge_index >> 32).astype(jnp.uint32)
        lo_t = jnp.transpose(lo, (0, 3, 1, 2))
        hi_t = jnp.transpose(hi, (0, 3, 1, 2))
        plane = _make_plane_kernel()
        olo = plane(lo_t)
        ohi = plane(hi_t)
        olo = jnp.transpose(olo, (0, 2, 3, 1))
        ohi = jnp.transpose(ohi, (0, 2, 3, 1))
        out = (ohi.astype(jnp.uint64) << jnp.uint64(32)) | olo.astype(
            jnp.uint64
        )
        return out.astype(jnp.int64)
    w = _KEEP
    out = _make_compact_kernel(w)(edge_index.reshape(_M * 2 * w))
    out = out.reshape(_B, _N, _P, _KEEP)
    return out.astype(edge_index.dtype)
